# Initial kernel scaffold; baseline (speedup 1.0000x reference)
import jax
import jax.numpy as jnp
from jax.experimental import pallas as pl


def kernel(x, edge_index, batch, edge_attr, params):
    raise NotImplementedError("write your pallas kernel here")


# speedup vs baseline: 17.9722x; 17.9722x over previous
import jax
import jax.numpy as jnp
from jax import lax
from jax.experimental import pallas as pl
from jax.experimental.pallas import tpu as pltpu
from jax.experimental.pallas import tpu_sc as plsc

N = 10000
NP = 10240
E = 320000
F = 128
ED = 16
C = 128
NG = 64
NGP = 128
TASK_HIDDEN = 64
ODIMS = (3, 3, 3, 4, 1, 1, 8)
OPAD = 8

NC = 2
NS = 16
NW = NC * NS
L = 16
EPW = E // NW
NH = NP // 2
NPT2 = NH // NS
RB = 1024
NB = NP // RB
NBH = NH // RB
K = 80
NCHUNK = EPW // K
SEL = EPW + K

_f32 = jnp.float32
_i32 = jnp.int32


def _lrelu16(v):
    return jnp.where(v >= 0.0, v, 0.2 * v)


def _proj_body(x_ref, w_ref, a_ref, xs_ref, asd_ref):
    xs = jnp.dot(x_ref[...], w_ref[...], preferred_element_type=_f32)
    xs_ref[...] = xs
    asd_ref[...] = jnp.dot(xs, a_ref[...], preferred_element_type=_f32)


def _proj(x, W, A8):
    return pl.pallas_call(
        _proj_body,
        grid=(NB,),
        in_specs=[
            pl.BlockSpec((RB, x.shape[1]), lambda i: (i, 0)),
            pl.BlockSpec(W.shape, lambda i: (0, 0)),
            pl.BlockSpec(A8.shape, lambda i: (0, 0)),
        ],
        out_specs=[
            pl.BlockSpec((RB, W.shape[1]), lambda i: (i, 0)),
            pl.BlockSpec((RB, A8.shape[1]), lambda i: (i, 0)),
        ],
        out_shape=[
            jax.ShapeDtypeStruct((NP, W.shape[1]), _f32),
            jax.ShapeDtypeStruct((NP, A8.shape[1]), _f32),
        ],
    )(x, W, A8)


def _eproj_body(ea_ref, v_ref, out_ref):
    out_ref[...] = jnp.dot(ea_ref[...], v_ref[...], preferred_element_type=_f32)


def _eproj(ea, V8, rb):
    grid = E // rb
    return pl.pallas_call(
        _eproj_body,
        grid=(grid,),
        in_specs=[
            pl.BlockSpec((rb, ED), lambda i: (i, 0)),
            pl.BlockSpec((ED, 8), lambda i: (0, 0)),
        ],
        out_specs=pl.BlockSpec((rb, 8), lambda i: (i, 0)),
        out_shape=jax.ShapeDtypeStruct((E, 8), _f32),
    )(ea, V8)


def _sc_layer1_body(src_h, dst_h, ae_h, asrc_h, adst_h, xs1_h,
                    num_h, den_h, deg_h, s1_h,
                    asrc_t, adst_t, deg_t, s1_t, den_t,
                    srcb, dstb, aeb, sel_src, sel_w, sel_dst2, rowbuf, zbuf,
                    acc, sem):
    c = lax.axis_index("c")
    s = lax.axis_index("s")
    w_id = s * NC + c

    z16f = jnp.zeros((L,), _f32)

    def _zb(i, _):
        for r in range(C // L):
            zbuf[i, pl.ds(r * L, L)] = z16f
        return 0
    lax.fori_loop(0, 16, _zb, 0)

    pltpu.sync_copy(asrc_h, asrc_t)
    pltpu.sync_copy(adst_h, adst_t)

    z16i = jnp.zeros((L,), _i32)

    def _zi(i, _):
        sel_src[pl.ds(i * L, L)] = z16i
        return 0
    lax.fori_loop(0, SEL // L, _zi, 0)

    def _zd(i, _):
        for r in range(K // L):
            sel_dst2[i, pl.ds(r * L, L)] = z16i
        return 0
    lax.fori_loop(0, SEL // K, _zd, 0)

    ones16 = jnp.full((L,), 1.0, _f32)

    for h in range(2):
        lo = h * NH

        def _za(t, _):
            pltpu.sync_copy(zbuf, acc.at[pl.ds(s * NPT2 + t * 16, 16)])
            return 0
        lax.fori_loop(0, NPT2 // 16, _za, 0)

        def _zt(i, _):
            deg_t[pl.ds(i * L, L)] = z16f
            s1_t[pl.ds(i * L, L)] = z16f
            den_t[pl.ds(i * L, L)] = z16f
            return 0
        lax.fori_loop(0, NH // L, _zt, 0)

        def _zw(i, _):
            sel_w[pl.ds(i * L, L)] = z16f
            return 0
        lax.fori_loop(0, SEL // L, _zw, 0)
        plsc.subcore_barrier()

        def scan(i, cnt):
            base = w_id * EPW + i * K
            pltpu.sync_copy(src_h.at[pl.ds(base, K)], srcb)
            pltpu.sync_copy(dst_h.at[pl.ds(base, K)], dstb)
            pltpu.sync_copy(ae_h.at[pl.ds(base, K)], aeb)
            for j in range(K // L):
                sv = srcb[pl.ds(j * L, L)]
                dv = dstb[pl.ds(j * L, L)]
                av = aeb[pl.ds(j * L, L)]
                a_s = plsc.load_gather(asrc_t, [sv])
                a_d = plsc.load_gather(adst_t, [dv])
                wv = jnp.exp(_lrelu16(a_s + a_d + av))
                if h == 0:
                    msk = dv < NH
                else:
                    msk = dv >= NH
                dloc = jnp.where(msk, dv - lo, 0)
                plsc.addupdate_scatter(deg_t, [dloc], ones16, mask=msk)
                plsc.addupdate_scatter(s1_t, [dloc], av, mask=msk)
                plsc.addupdate_scatter(den_t, [dloc], wv, mask=msk)
                mi = msk.astype(_i32)
                pos = cnt + plsc.cumsum(mi) - mi
                plsc.store_scatter(sel_src, [pos], sv, mask=msk)
                plsc.store_scatter(sel_w, [pos], wv, mask=msk)
                plsc.store_scatter(sel_dst2, [pos // K, pos - (pos // K) * K],
                                   dloc, mask=msk)
                cnt = cnt + jnp.sum(mi)
            return cnt

        cnt = lax.fori_loop(0, NCHUNK, scan, jnp.zeros((), _i32))
        ntrip = (cnt + (K - 1)) // K

        def drain(t, _):
            pltpu.async_copy(xs1_h.at[sel_src.at[pl.ds(t * K, K)]], rowbuf,
                             sem).wait()

            def scale(g, _):
                wv = sel_w[pl.ds(t * K + g * L, L)]
                for i2 in range(L):
                    w_sc = wv[i2]
                    for r in range(C // L):
                        rowbuf[g * L + i2, pl.ds(r * L, L)] = (
                            rowbuf[g * L + i2, pl.ds(r * L, L)] * w_sc)
                return 0
            lax.fori_loop(0, K // L, scale, 0)
            pltpu.sync_copy(rowbuf, acc.at[sel_dst2.at[t]], add=True)
            return 0

        lax.fori_loop(0, ntrip, drain, 0)

        for t in range(NBH):
            pltpu.sync_copy(deg_t.at[pl.ds(t * RB, RB)],
                            deg_h.at[h * NBH + t, w_id])
            pltpu.sync_copy(s1_t.at[pl.ds(t * RB, RB)],
                            s1_h.at[h * NBH + t, w_id])
            pltpu.sync_copy(den_t.at[pl.ds(t * RB, RB)],
                            den_h.at[h * NBH + t, w_id])

        plsc.subcore_barrier()

        pltpu.sync_copy(acc.at[pl.ds(s * NPT2, NPT2)],
                        num_h.at[c, pl.ds(lo + s * NPT2, NPT2)])
        plsc.subcore_barrier()


def _sc_layer1(src, dst, ae1, asrc1, adst1, xs1):
    mesh = plsc.VectorSubcoreMesh(core_axis_name="c", subcore_axis_name="s")
    fn = pl.kernel(
        _sc_layer1_body,
        out_type=[
            jax.ShapeDtypeStruct((NC, NP, C), _f32),
            jax.ShapeDtypeStruct((NB, NW, RB), _f32),
            jax.ShapeDtypeStruct((NB, NW, RB), _f32),
            jax.ShapeDtypeStruct((NB, NW, RB), _f32),
        ],
        mesh=mesh,
        scratch_types=[
            pltpu.VMEM((NP,), _f32),
            pltpu.VMEM((NP,), _f32),
            pltpu.VMEM((NH,), _f32),
            pltpu.VMEM((NH,), _f32),
            pltpu.VMEM((NH,), _f32),
            pltpu.VMEM((K,), _i32),
            pltpu.VMEM((K,), _i32),
            pltpu.VMEM((K,), _f32),
            pltpu.VMEM((SEL,), _i32),
            pltpu.VMEM((SEL,), _f32),
            pltpu.VMEM((SEL // K, K), _i32),
            pltpu.VMEM((K, C), _f32),
            pltpu.VMEM((16, C), _f32),
            pltpu.VMEM_SHARED((NH, C), _f32),
            pltpu.SemaphoreType.DMA,
        ],
        compiler_params=pltpu.CompilerParams(needs_layout_passes=False),
    )
    return fn(src, dst, ae1, asrc1, adst1, xs1)


def _combine_body(num_ref, den_ref, deg_ref, s1_ref, asd_ref, xs1_ref,
                  w2_ref, a2_ref, b1_ref, xs2_ref, asd2_ref):
    num = num_ref[0] + num_ref[1]
    den = jnp.sum(den_ref[0], axis=0)
    deg = jnp.sum(deg_ref[0], axis=0)
    s1 = jnp.sum(s1_ref[0], axis=0)
    asrc = asd_ref[:, 0]
    adst = asd_ref[:, 1]
    al = asrc + adst + s1 / jnp.maximum(deg, 1.0)
    wl = jnp.exp(jnp.where(al >= 0.0, al, 0.2 * al))
    xs1 = xs1_ref[...]
    h1 = (num + wl[:, None] * xs1) / (den + wl)[:, None] + b1_ref[0]
    h1 = jnp.maximum(h1, 0.0)
    xs2 = jnp.dot(h1, w2_ref[...], preferred_element_type=_f32)
    xs2_ref[...] = xs2
    asd2_ref[...] = jnp.dot(xs2, a2_ref[...], preferred_element_type=_f32)


def _combine(num_p, den_p, deg_p, s1_p, asd1, xs1, W2, A2, b1):
    return pl.pallas_call(
        _combine_body,
        grid=(NB,),
        in_specs=[
            pl.BlockSpec((NC, RB, C), lambda i: (0, i, 0)),
            pl.BlockSpec((1, NW, RB), lambda i: (i, 0, 0)),
            pl.BlockSpec((1, NW, RB), lambda i: (i, 0, 0)),
            pl.BlockSpec((1, NW, RB), lambda i: (i, 0, 0)),
            pl.BlockSpec((RB, 8), lambda i: (i, 0)),
            pl.BlockSpec((RB, C), lambda i: (i, 0)),
            pl.BlockSpec((C, C), lambda i: (0, 0)),
            pl.BlockSpec((C, 8), lambda i: (0, 0)),
            pl.BlockSpec((1, C), lambda i: (0, 0)),
        ],
        out_specs=[
            pl.BlockSpec((RB, C), lambda i: (i, 0)),
            pl.BlockSpec((RB, 8), lambda i: (i, 0)),
        ],
        out_shape=[
            jax.ShapeDtypeStruct((NP, C), _f32),
            jax.ShapeDtypeStruct((NP, 8), _f32),
        ],
    )(num_p, den_p, deg_p, s1_p, asd1, xs1, W2, A2, b1)


def _sc_layer2_body(src_h, dst_h, ae_h, asrc_h, adst_h, batch_h, x_h, xs2_h,
                    acc2_h, den2_h, deg2_h, s2_h, xroot2_h, asroot_h, adroot_h,
                    remap_h, xroot_h,
                    member_t, asrc_t, adst_t, sel_src, sel_slot, sel_w,
                    srcb, dstb, aeb, rowbuf, acc2f,
                    den2b, deg2b, s2b, n0b, rootb, member_s, sem):
    c = lax.axis_index("c")
    s = lax.axis_index("s")
    w_id = s * NC + c

    z16f = jnp.zeros((L,), _f32)
    z16i = jnp.zeros((L,), _i32)
    pltpu.sync_copy(asrc_h, asrc_t)
    pltpu.sync_copy(adst_h, adst_t)

    @pl.when(s == 0)
    def _():
        ones16i = jnp.full((L,), 1, _i32)
        for k in range(NG // L):
            member_t[pl.ds(k * L, L)] = z16i

        def hist(i, _):
            pltpu.sync_copy(batch_h.at[pl.ds(i * K, K)], srcb)
            for j in range(K // L):
                bv = srcb[pl.ds(j * L, L)]
                plsc.addupdate_scatter(member_t, [bv], ones16i)
            return 0
        lax.fori_loop(0, NP // K, hist, 0)

        carry = jnp.zeros((), _i32)
        for k in range(NG // L):
            hv = member_t[pl.ds(k * L, L)]
            cs = plsc.cumsum(hv)
            n0b[pl.ds(k * L, L)] = cs - hv + carry
            carry = carry + jnp.sum(hv)
        for k in range(NG // L, NGP // L):
            n0b[pl.ds(k * L, L)] = z16i

        neg16 = jnp.full((L,), -1, _i32)

        def fill(i, _):
            member_t[pl.ds(i * L, L)] = neg16
            return 0
        lax.fori_loop(0, NP // L, fill, 0)
        iota16 = lax.iota(_i32, L)
        for k in range(NG // L):
            idxv = n0b[pl.ds(k * L, L)]
            plsc.store_scatter(member_t, [idxv], iota16 + k * L)
        pltpu.sync_copy(member_t, member_s)

    plsc.subcore_barrier()
    pltpu.sync_copy(member_s, member_t)

    def _zs(i, _):
        sel_src[pl.ds(i * L, L)] = z16i
        sel_slot[pl.ds(i * L, L)] = z16i
        sel_w[pl.ds(i * L, L)] = z16f
        return 0
    lax.fori_loop(0, SEL // L, _zs, 0)
    for k in range(NGP // L):
        den2b[pl.ds(k * L, L)] = z16f
        deg2b[pl.ds(k * L, L)] = z16f
        s2b[pl.ds(k * L, L)] = z16f

    def _za(i, _):
        for r in range(C // L):
            acc2f[i, pl.ds(r * L, L)] = z16f
        return 0
    lax.fori_loop(0, NG, _za, 0)

    ones16 = jnp.full((L,), 1.0, _f32)

    def scan(i, cnt):
        base = w_id * EPW + i * K
        pltpu.sync_copy(src_h.at[pl.ds(base, K)], srcb)
        pltpu.sync_copy(dst_h.at[pl.ds(base, K)], dstb)
        pltpu.sync_copy(ae_h.at[pl.ds(base, K)], aeb)
        for j in range(K // L):
            sv = srcb[pl.ds(j * L, L)]
            dv = dstb[pl.ds(j * L, L)]
            av = aeb[pl.ds(j * L, L)]
            mv = plsc.load_gather(member_t, [dv])
            msk = mv >= 0
            midx = jnp.where(msk, mv, 0)
            a_s = plsc.load_gather(asrc_t, [sv])
            a_d = plsc.load_gather(adst_t, [dv])
            wv = jnp.exp(_lrelu16(a_s + a_d + av))
            plsc.addupdate_scatter(deg2b, [midx], ones16, mask=msk)
            plsc.addupdate_scatter(s2b, [midx], av, mask=msk)
            plsc.addupdate_scatter(den2b, [midx], wv, mask=msk)
            mi = msk.astype(_i32)
            pos = cnt + plsc.cumsum(mi) - mi
            plsc.store_scatter(sel_src, [pos], sv, mask=msk)
            plsc.store_scatter(sel_slot, [pos], midx, mask=msk)
            plsc.store_scatter(sel_w, [pos], wv, mask=msk)
            cnt = cnt + jnp.sum(mi)
        return cnt

    cnt = lax.fori_loop(0, NCHUNK, scan, jnp.zeros((), _i32))

    ntrip = (cnt + (K - 1)) // K
    iota16 = lax.iota(_i32, L)

    def gather_acc(t, _):
        pltpu.async_copy(xs2_h.at[sel_src.at[pl.ds(t * K, K)]], rowbuf,
                         sem).wait()

        def row(g, _):
            wv = sel_w[pl.ds(t * K + g * L, L)]
            slv = sel_slot[pl.ds(t * K + g * L, L)]
            for i2 in range(L):
                w_sc = wv[i2]
                slotv = jnp.full((L,), slv[i2], _i32)
                for r in range(C // L):
                    v = rowbuf[g * L + i2, pl.ds(r * L, L)] * w_sc
                    plsc.addupdate_scatter(acc2f, [slotv, r * L + iota16], v)
            return 0
        lax.fori_loop(0, K // L, row, 0)
        return 0

    lax.fori_loop(0, ntrip, gather_acc, 0)

    pltpu.sync_copy(acc2f, acc2_h.at[w_id])
    pltpu.sync_copy(den2b, den2_h.at[w_id])
    pltpu.sync_copy(deg2b, deg2_h.at[w_id])
    pltpu.sync_copy(s2b, s2_h.at[w_id])

    @pl.when(jnp.logical_and(c == 0, s == 0))
    def _():
        pltpu.async_copy(x_h.at[n0b.at[pl.ds(0, NG)]], rowbuf.at[pl.ds(0, NG)],
                         sem).wait()
        pltpu.sync_copy(rowbuf.at[pl.ds(0, NG)], xroot_h)
        pltpu.async_copy(xs2_h.at[n0b.at[pl.ds(0, NG)]], rowbuf.at[pl.ds(0, NG)],
                         sem).wait()
        pltpu.sync_copy(rowbuf.at[pl.ds(0, NG)], xroot2_h)
        z16f_ = jnp.zeros((L,), _f32)
        for k in range(NG // L):
            idxv = n0b[pl.ds(k * L, L)]
            rootb[pl.ds(k * L, L)] = plsc.load_gather(asrc_t, [idxv])
        for k in range(NG // L, NGP // L):
            rootb[pl.ds(k * L, L)] = z16f_
        pltpu.sync_copy(rootb, asroot_h)
        for k in range(NG // L):
            idxv = n0b[pl.ds(k * L, L)]
            rootb[pl.ds(k * L, L)] = plsc.load_gather(adst_t, [idxv])
        pltpu.sync_copy(rootb, adroot_h)
        for k in range(NG // L):
            idxv = n0b[pl.ds(k * L, L)]
            mv = plsc.load_gather(member_t, [idxv])
            n0b[pl.ds(k * L, L)] = mv
        pltpu.sync_copy(n0b, remap_h)


def _sc_layer2(src, dst, ae2, asrc2, adst2, batch, x, xs2):
    mesh = plsc.VectorSubcoreMesh(core_axis_name="c", subcore_axis_name="s")
    fn = pl.kernel(
        _sc_layer2_body,
        out_type=[
            jax.ShapeDtypeStruct((NW, NG, C), _f32),
            jax.ShapeDtypeStruct((NW, NGP), _f32),
            jax.ShapeDtypeStruct((NW, NGP), _f32),
            jax.ShapeDtypeStruct((NW, NGP), _f32),
            jax.ShapeDtypeStruct((NG, C), _f32),
            jax.ShapeDtypeStruct((NGP,), _f32),
            jax.ShapeDtypeStruct((NGP,), _f32),
            jax.ShapeDtypeStruct((NGP,), _i32),
            jax.ShapeDtypeStruct((NG, C), _f32),
        ],
        mesh=mesh,
        scratch_types=[
            pltpu.VMEM((NP,), _i32),
            pltpu.VMEM((NP,), _f32),
            pltpu.VMEM((NP,), _f32),
            pltpu.VMEM((SEL,), _i32),
            pltpu.VMEM((SEL,), _i32),
            pltpu.VMEM((SEL,), _f32),
            pltpu.VMEM((K,), _i32),
            pltpu.VMEM((K,), _i32),
            pltpu.VMEM((K,), _f32),
            pltpu.VMEM((K, C), _f32),
            pltpu.VMEM((NG, C), _f32),
            pltpu.VMEM((NGP,), _f32),
            pltpu.VMEM((NGP,), _f32),
            pltpu.VMEM((NGP,), _f32),
            pltpu.VMEM((NGP,), _i32),
            pltpu.VMEM((NGP,), _f32),
            pltpu.VMEM_SHARED((NP,), _i32),
            pltpu.SemaphoreType.DMA,
        ],
        compiler_params=pltpu.CompilerParams(needs_layout_passes=False),
    )
    return fn(src, dst, ae2, asrc2, adst2, batch, x, xs2)


def _heads_body(acc2_ref, den2_ref, deg2_ref, s2_ref, asr_ref, adr_ref,
                remap_ref, xr2_ref, xroot_ref, b2_ref, sw_ref, sb_ref,
                w1_ref, b1c_ref, w2_ref, b2c_ref, out_ref):
    num2 = jnp.sum(acc2_ref[...], axis=0)
    den2 = jnp.sum(den2_ref[...], axis=0)[:NG]
    deg2 = jnp.sum(deg2_ref[...], axis=0)[:NG]
    s2 = jnp.sum(s2_ref[...], axis=0)[:NG]
    asr = asr_ref[0, :NG]
    adr = adr_ref[0, :NG]
    al = asr + adr + s2 / jnp.maximum(deg2, 1.0)
    wl = jnp.exp(jnp.where(al >= 0.0, al, 0.2 * al))
    xr2 = xr2_ref[...]
    h2s = (num2 + wl[:, None] * xr2) / (den2 + wl)[:, None] + b2_ref[0]
    remap = remap_ref[:NG]
    onehot = (lax.broadcasted_iota(_i32, (NG, NG), 1) == remap).astype(_f32)
    h2 = jnp.dot(onehot, h2s, preferred_element_type=_f32)
    gin = jnp.concatenate([h2, xroot_ref[:, :ED]], axis=1)
    g = jnp.dot(gin, sw_ref[...], preferred_element_type=_f32) + sb_ref[0]
    g = jnp.maximum(g, 0.0)
    t = jnp.dot(g, w1_ref[...], preferred_element_type=_f32) + b1c_ref[0]
    t = jnp.maximum(t, 0.0)
    out_ref[...] = jnp.dot(t, w2_ref[...], preferred_element_type=_f32) + b2c_ref[0]


def _heads(acc2_p, den2_p, deg2_p, s2_p, asroot, adroot, remap, xroot2, xroot,
           b2, sW, sb, W1c, b1c, W2bd, b2c):
    nh = len(ODIMS)
    return pl.pallas_call(
        _heads_body,
        out_shape=jax.ShapeDtypeStruct((NG, nh * OPAD), _f32),
    )(acc2_p, den2_p, deg2_p, s2_p, asroot, adroot, remap, xroot2, xroot,
      b2, sW, sb, W1c, b1c, W2bd, b2c)


@jax.jit
def kernel(x, edge_index, batch, edge_attr, params):
    src = edge_index[0].astype(_i32)
    dst = edge_index[1].astype(_i32)
    batch_p = jnp.concatenate(
        [batch.astype(_i32), jnp.full((NP - N,), NG - 1, _i32)])
    x_p = jnp.zeros((NP, F), _f32).at[:N].set(x)
    g1, g2 = params['gat']

    ve1 = g1['W_edge'] @ g1['att_edge'][0]
    ve2 = g2['W_edge'] @ g2['att_edge'][0]
    V8 = jnp.zeros((ED, 8), _f32).at[:, 0].set(ve1).at[:, 1].set(ve2)
    A1 = jnp.zeros((C, 8), _f32).at[:, 0].set(g1['att_src'][0]).at[:, 1].set(g1['att_dst'][0])
    A2 = jnp.zeros((C, 8), _f32).at[:, 0].set(g2['att_src'][0]).at[:, 1].set(g2['att_dst'][0])

    ae8 = _eproj(edge_attr, V8, 8000)
    ae1 = ae8[:, 0]
    ae2 = ae8[:, 1]

    xs1, asd1 = _proj(x_p, g1['W'], A1)
    asrc1 = asd1[:, 0]
    adst1 = asd1[:, 1]

    num_p, den_p, deg_p, s1_p = _sc_layer1(src, dst, ae1, asrc1, adst1, xs1)

    b1 = g1['bias'].reshape(1, C)
    xs2, asd2 = _combine(num_p, den_p, deg_p, s1_p, asd1, xs1, g2['W'], A2, b1)
    asrc2 = asd2[:, 0]
    adst2 = asd2[:, 1]

    (acc2_p, den2_p, deg2_p, s2_p, xroot2, asroot, adroot, remap,
     xroot) = _sc_layer2(src, dst, ae2, asrc2, adst2, batch_p, x_p, xs2)

    names = ['ptr_l1', 'ptr_l2', 'ptr_l3', 'leaf_category', 'leaf_signed',
             'leaf_floating', 'leaf_size']
    nh = len(names)
    W1c = jnp.concatenate([params['heads'][n][0][0] for n in names], axis=1)
    b1c = jnp.concatenate([params['heads'][n][0][1] for n in names]).reshape(1, -1)
    W2bd = jnp.zeros((nh * TASK_HIDDEN, nh * OPAD), _f32)
    b2c = jnp.zeros((1, nh * OPAD), _f32)
    for i, n in enumerate(names):
        W2h, b2h = params['heads'][n][1]
        od = ODIMS[i]
        W2bd = W2bd.at[i * TASK_HIDDEN:(i + 1) * TASK_HIDDEN,
                       i * OPAD:i * OPAD + od].set(W2h)
        b2c = b2c.at[0, i * OPAD:i * OPAD + od].set(b2h)
    (sW, sb) = params['shared'][0]

    out = _heads(acc2_p, den2_p, deg2_p, s2_p, asroot.reshape(1, NGP),
                 adroot.reshape(1, NGP), remap.reshape(NGP, 1),
                 xroot2, xroot, g2['bias'].reshape(1, C), sW,
                 sb.reshape(1, -1), W1c, b1c, W2bd, b2c)

    return tuple(out[:, i * OPAD:i * OPAD + od] for i, od in enumerate(ODIMS))


# speedup vs baseline: 23.2458x; 1.2934x over previous
import jax
import jax.numpy as jnp
from jax import lax
from jax.experimental import pallas as pl
from jax.experimental.pallas import tpu as pltpu
from jax.experimental.pallas import tpu_sc as plsc

N = 10000
NP = 10240
E = 320000
F = 128
ED = 16
C = 128
NG = 64
NGP = 128
TASK_HIDDEN = 64
ODIMS = (3, 3, 3, 4, 1, 1, 8)
OPAD = 8

NC = 2
NS = 16
NW = NC * NS
L = 16
EPW = E // NW
NH = NP // 2
NPT2 = NH // NS
RB = 1024
NB = NP // RB
NBH = NH // RB
K = 80
KS = 400
NCHUNK = EPW // KS
SEL = EPW + K

_f32 = jnp.float32
_i32 = jnp.int32


def _lrelu16(v):
    return jnp.where(v >= 0.0, v, 0.2 * v)


def _proj_body(x_ref, w_ref, a_ref, xs_ref, asd_ref):
    xs = jnp.dot(x_ref[...], w_ref[...], preferred_element_type=_f32)
    xs_ref[...] = xs
    asd_ref[...] = jnp.dot(xs, a_ref[...], preferred_element_type=_f32)


def _proj(x, W, A8):
    return pl.pallas_call(
        _proj_body,
        grid=(NB,),
        in_specs=[
            pl.BlockSpec((RB, x.shape[1]), lambda i: (i, 0)),
            pl.BlockSpec(W.shape, lambda i: (0, 0)),
            pl.BlockSpec(A8.shape, lambda i: (0, 0)),
        ],
        out_specs=[
            pl.BlockSpec((RB, W.shape[1]), lambda i: (i, 0)),
            pl.BlockSpec((RB, A8.shape[1]), lambda i: (i, 0)),
        ],
        out_shape=[
            jax.ShapeDtypeStruct((NP, W.shape[1]), _f32),
            jax.ShapeDtypeStruct((NP, A8.shape[1]), _f32),
        ],
    )(x, W, A8)


def _eproj_body(ea_ref, v_ref, out_ref):
    out_ref[...] = jnp.dot(ea_ref[...], v_ref[...], preferred_element_type=_f32)


def _eproj(ea, V8, rb):
    grid = E // rb
    return pl.pallas_call(
        _eproj_body,
        grid=(grid,),
        in_specs=[
            pl.BlockSpec((rb, ED), lambda i: (i, 0)),
            pl.BlockSpec((ED, 8), lambda i: (0, 0)),
        ],
        out_specs=pl.BlockSpec((rb, 8), lambda i: (i, 0)),
        out_shape=jax.ShapeDtypeStruct((E, 8), _f32),
    )(ea, V8)


def _sc_layer1_body(src_h, dst_h, ae_h, asrc_h, adst_h, xs1_h,
                    num_h, den_h, deg_h, s1_h,
                    asrc_t, adst_t, deg_t, s1_t, den_t,
                    srcb, dstb, aeb, sel_src, sel_w, sel_dst2, rowbuf, zbuf,
                    acc, sem):
    c = lax.axis_index("c")
    s = lax.axis_index("s")
    w_id = s * NC + c

    z16f = jnp.zeros((L,), _f32)

    def _zb(i, _):
        for r in range(C // L):
            zbuf[i, pl.ds(r * L, L)] = z16f
        return 0
    lax.fori_loop(0, 16, _zb, 0)

    pltpu.sync_copy(asrc_h, asrc_t)
    pltpu.sync_copy(adst_h, adst_t)

    z16i = jnp.zeros((L,), _i32)

    def _zi(i, _):
        sel_src[pl.ds(i * L, L)] = z16i
        return 0
    lax.fori_loop(0, SEL // L, _zi, 0)

    def _zd(i, _):
        for r in range(K // L):
            sel_dst2[i, pl.ds(r * L, L)] = z16i
        return 0
    lax.fori_loop(0, SEL // K, _zd, 0)

    ones16 = jnp.full((L,), 1.0, _f32)

    for h in range(2):
        lo = h * NH

        def _za(t, _):
            pltpu.async_copy(zbuf, acc.at[pl.ds(s * NPT2 + t * 16, 16)], sem)
            return 0
        lax.fori_loop(0, NPT2 // 16, _za, 0)

        def _zaw(t, _):
            pltpu.make_async_copy(zbuf, acc.at[pl.ds(s * NPT2 + t * 16, 16)],
                                  sem).wait()
            return 0
        lax.fori_loop(0, NPT2 // 16, _zaw, 0)

        def _zt(i, _):
            deg_t[pl.ds(i * L, L)] = z16f
            s1_t[pl.ds(i * L, L)] = z16f
            den_t[pl.ds(i * L, L)] = z16f
            return 0
        lax.fori_loop(0, NH // L, _zt, 0)

        def _zw(i, _):
            sel_w[pl.ds(i * L, L)] = z16f
            return 0
        lax.fori_loop(0, SEL // L, _zw, 0)
        plsc.subcore_barrier()

        def scan(i, cnt):
            base = w_id * EPW + i * KS
            pltpu.sync_copy(src_h.at[pl.ds(base, KS)], srcb)
            pltpu.sync_copy(dst_h.at[pl.ds(base, KS)], dstb)
            pltpu.sync_copy(ae_h.at[pl.ds(base, KS)], aeb)
            for j in range(KS // L):
                sv = srcb[pl.ds(j * L, L)]
                dv = dstb[pl.ds(j * L, L)]
                av = aeb[pl.ds(j * L, L)]
                a_s = plsc.load_gather(asrc_t, [sv])
                a_d = plsc.load_gather(adst_t, [dv])
                wv = jnp.exp(_lrelu16(a_s + a_d + av))
                if h == 0:
                    msk = dv < NH
                else:
                    msk = dv >= NH
                dloc = jnp.where(msk, dv - lo, 0)
                plsc.addupdate_scatter(deg_t, [dloc], ones16, mask=msk)
                plsc.addupdate_scatter(s1_t, [dloc], av, mask=msk)
                plsc.addupdate_scatter(den_t, [dloc], wv, mask=msk)
                mi = msk.astype(_i32)
                pos = cnt + plsc.cumsum(mi) - mi
                plsc.store_scatter(sel_src, [pos], sv, mask=msk)
                plsc.store_scatter(sel_w, [pos], wv, mask=msk)
                plsc.store_scatter(sel_dst2, [pos // K, pos - (pos // K) * K],
                                   dloc, mask=msk)
                cnt = cnt + jnp.sum(mi)
            return cnt

        cnt = lax.fori_loop(0, NCHUNK, scan, jnp.zeros((), _i32))
        ntrip = (cnt + (K - 1)) // K

        def drain(t, _):
            pltpu.async_copy(xs1_h.at[sel_src.at[pl.ds(t * K, K)]], rowbuf,
                             sem).wait()

            def scale(g, _):
                wv = sel_w[pl.ds(t * K + g * L, L)]
                for i2 in range(L):
                    w_sc = wv[i2]
                    for r in range(C // L):
                        rowbuf[g * L + i2, pl.ds(r * L, L)] = (
                            rowbuf[g * L + i2, pl.ds(r * L, L)] * w_sc)
                return 0
            lax.fori_loop(0, K // L, scale, 0)
            pltpu.sync_copy(rowbuf, acc.at[sel_dst2.at[t]], add=True)
            return 0

        lax.fori_loop(0, ntrip, drain, 0)

        for t in range(NBH):
            pltpu.sync_copy(deg_t.at[pl.ds(t * RB, RB)],
                            deg_h.at[h * NBH + t, w_id])
            pltpu.sync_copy(s1_t.at[pl.ds(t * RB, RB)],
                            s1_h.at[h * NBH + t, w_id])
            pltpu.sync_copy(den_t.at[pl.ds(t * RB, RB)],
                            den_h.at[h * NBH + t, w_id])

        plsc.subcore_barrier()

        pltpu.sync_copy(acc.at[pl.ds(s * NPT2, NPT2)],
                        num_h.at[c, pl.ds(lo + s * NPT2, NPT2)])
        plsc.subcore_barrier()


def _sc_layer1(src, dst, ae1, asrc1, adst1, xs1):
    mesh = plsc.VectorSubcoreMesh(core_axis_name="c", subcore_axis_name="s")
    fn = pl.kernel(
        _sc_layer1_body,
        out_type=[
            jax.ShapeDtypeStruct((NC, NP, C), _f32),
            jax.ShapeDtypeStruct((NB, NW, RB), _f32),
            jax.ShapeDtypeStruct((NB, NW, RB), _f32),
            jax.ShapeDtypeStruct((NB, NW, RB), _f32),
        ],
        mesh=mesh,
        scratch_types=[
            pltpu.VMEM((NP,), _f32),
            pltpu.VMEM((NP,), _f32),
            pltpu.VMEM((NH,), _f32),
            pltpu.VMEM((NH,), _f32),
            pltpu.VMEM((NH,), _f32),
            pltpu.VMEM((KS,), _i32),
            pltpu.VMEM((KS,), _i32),
            pltpu.VMEM((KS,), _f32),
            pltpu.VMEM((SEL,), _i32),
            pltpu.VMEM((SEL,), _f32),
            pltpu.VMEM((SEL // K, K), _i32),
            pltpu.VMEM((K, C), _f32),
            pltpu.VMEM((16, C), _f32),
            pltpu.VMEM_SHARED((NH, C), _f32),
            pltpu.SemaphoreType.DMA,
        ],
        compiler_params=pltpu.CompilerParams(needs_layout_passes=False),
    )
    return fn(src, dst, ae1, asrc1, adst1, xs1)


def _combine_body(num_ref, den_ref, deg_ref, s1_ref, asd_ref, xs1_ref,
                  w2_ref, a2_ref, b1_ref, xs2_ref, asd2_ref):
    num = num_ref[0] + num_ref[1]
    den = jnp.sum(den_ref[0], axis=0)
    deg = jnp.sum(deg_ref[0], axis=0)
    s1 = jnp.sum(s1_ref[0], axis=0)
    asrc = asd_ref[:, 0]
    adst = asd_ref[:, 1]
    al = asrc + adst + s1 / jnp.maximum(deg, 1.0)
    wl = jnp.exp(jnp.where(al >= 0.0, al, 0.2 * al))
    xs1 = xs1_ref[...]
    h1 = (num + wl[:, None] * xs1) / (den + wl)[:, None] + b1_ref[0]
    h1 = jnp.maximum(h1, 0.0)
    xs2 = jnp.dot(h1, w2_ref[...], preferred_element_type=_f32)
    xs2_ref[...] = xs2
    asd2_ref[...] = jnp.dot(xs2, a2_ref[...], preferred_element_type=_f32)


def _combine(num_p, den_p, deg_p, s1_p, asd1, xs1, W2, A2, b1):
    return pl.pallas_call(
        _combine_body,
        grid=(NB,),
        in_specs=[
            pl.BlockSpec((NC, RB, C), lambda i: (0, i, 0)),
            pl.BlockSpec((1, NW, RB), lambda i: (i, 0, 0)),
            pl.BlockSpec((1, NW, RB), lambda i: (i, 0, 0)),
            pl.BlockSpec((1, NW, RB), lambda i: (i, 0, 0)),
            pl.BlockSpec((RB, 8), lambda i: (i, 0)),
            pl.BlockSpec((RB, C), lambda i: (i, 0)),
            pl.BlockSpec((C, C), lambda i: (0, 0)),
            pl.BlockSpec((C, 8), lambda i: (0, 0)),
            pl.BlockSpec((1, C), lambda i: (0, 0)),
        ],
        out_specs=[
            pl.BlockSpec((RB, C), lambda i: (i, 0)),
            pl.BlockSpec((RB, 8), lambda i: (i, 0)),
        ],
        out_shape=[
            jax.ShapeDtypeStruct((NP, C), _f32),
            jax.ShapeDtypeStruct((NP, 8), _f32),
        ],
    )(num_p, den_p, deg_p, s1_p, asd1, xs1, W2, A2, b1)


def _sc_layer2_body(src_h, dst_h, ae_h, asrc_h, adst_h, batch_h, x_h, xs2_h,
                    acc2_h, den2_h, deg2_h, s2_h, xroot2_h, asroot_h, adroot_h,
                    remap_h, xroot_h,
                    member_t, asrc_t, adst_t, sel_src, sel_slot, sel_w,
                    srcb, dstb, aeb, rowbuf, acc2f,
                    den2b, deg2b, s2b, n0b, rootb, member_s, sem):
    c = lax.axis_index("c")
    s = lax.axis_index("s")
    w_id = s * NC + c

    z16f = jnp.zeros((L,), _f32)
    z16i = jnp.zeros((L,), _i32)
    pltpu.sync_copy(asrc_h, asrc_t)
    pltpu.sync_copy(adst_h, adst_t)

    @pl.when(s == 0)
    def _():
        ones16i = jnp.full((L,), 1, _i32)
        for k in range(NG // L):
            member_t[pl.ds(k * L, L)] = z16i

        def hist(i, _):
            pltpu.sync_copy(batch_h.at[pl.ds(i * 320, 320)],
                            srcb.at[pl.ds(0, 320)])
            for j in range(320 // L):
                bv = srcb[pl.ds(j * L, L)]
                plsc.addupdate_scatter(member_t, [bv], ones16i)
            return 0
        lax.fori_loop(0, NP // 320, hist, 0)

        carry = jnp.zeros((), _i32)
        for k in range(NG // L):
            hv = member_t[pl.ds(k * L, L)]
            cs = plsc.cumsum(hv)
            n0b[pl.ds(k * L, L)] = cs - hv + carry
            carry = carry + jnp.sum(hv)
        for k in range(NG // L, NGP // L):
            n0b[pl.ds(k * L, L)] = z16i

        neg16 = jnp.full((L,), -1, _i32)

        def fill(i, _):
            member_t[pl.ds(i * L, L)] = neg16
            return 0
        lax.fori_loop(0, NP // L, fill, 0)
        iota16 = lax.iota(_i32, L)
        for k in range(NG // L):
            idxv = n0b[pl.ds(k * L, L)]
            plsc.store_scatter(member_t, [idxv], iota16 + k * L)
        pltpu.sync_copy(member_t, member_s)

    plsc.subcore_barrier()
    pltpu.sync_copy(member_s, member_t)

    def _zs(i, _):
        sel_src[pl.ds(i * L, L)] = z16i
        sel_slot[pl.ds(i * L, L)] = z16i
        sel_w[pl.ds(i * L, L)] = z16f
        return 0
    lax.fori_loop(0, SEL // L, _zs, 0)
    for k in range(NGP // L):
        den2b[pl.ds(k * L, L)] = z16f
        deg2b[pl.ds(k * L, L)] = z16f
        s2b[pl.ds(k * L, L)] = z16f

    def _za(i, _):
        for r in range(C // L):
            acc2f[i, pl.ds(r * L, L)] = z16f
        return 0
    lax.fori_loop(0, NG, _za, 0)

    ones16 = jnp.full((L,), 1.0, _f32)

    def scan(i, cnt):
        base = w_id * EPW + i * KS
        pltpu.sync_copy(src_h.at[pl.ds(base, KS)], srcb)
        pltpu.sync_copy(dst_h.at[pl.ds(base, KS)], dstb)
        pltpu.sync_copy(ae_h.at[pl.ds(base, KS)], aeb)
        for j in range(KS // L):
            sv = srcb[pl.ds(j * L, L)]
            dv = dstb[pl.ds(j * L, L)]
            av = aeb[pl.ds(j * L, L)]
            mv = plsc.load_gather(member_t, [dv])
            msk = mv >= 0
            midx = jnp.where(msk, mv, 0)
            a_s = plsc.load_gather(asrc_t, [sv])
            a_d = plsc.load_gather(adst_t, [dv])
            wv = jnp.exp(_lrelu16(a_s + a_d + av))
            plsc.addupdate_scatter(deg2b, [midx], ones16, mask=msk)
            plsc.addupdate_scatter(s2b, [midx], av, mask=msk)
            plsc.addupdate_scatter(den2b, [midx], wv, mask=msk)
            mi = msk.astype(_i32)
            pos = cnt + plsc.cumsum(mi) - mi
            plsc.store_scatter(sel_src, [pos], sv, mask=msk)
            plsc.store_scatter(sel_slot, [pos], midx, mask=msk)
            plsc.store_scatter(sel_w, [pos], wv, mask=msk)
            cnt = cnt + jnp.sum(mi)
        return cnt

    cnt = lax.fori_loop(0, NCHUNK, scan, jnp.zeros((), _i32))

    ntrip = (cnt + (K - 1)) // K
    iota16 = lax.iota(_i32, L)

    def gather_acc(t, _):
        pltpu.async_copy(xs2_h.at[sel_src.at[pl.ds(t * K, K)]], rowbuf,
                         sem).wait()

        def row(g, _):
            wv = sel_w[pl.ds(t * K + g * L, L)]
            slv = sel_slot[pl.ds(t * K + g * L, L)]
            for i2 in range(L):
                w_sc = wv[i2]
                slotv = jnp.full((L,), slv[i2], _i32)
                for r in range(C // L):
                    v = rowbuf[g * L + i2, pl.ds(r * L, L)] * w_sc
                    plsc.addupdate_scatter(acc2f, [slotv, r * L + iota16], v)
            return 0
        lax.fori_loop(0, K // L, row, 0)
        return 0

    lax.fori_loop(0, ntrip, gather_acc, 0)

    pltpu.sync_copy(acc2f, acc2_h.at[w_id])
    pltpu.sync_copy(den2b, den2_h.at[w_id])
    pltpu.sync_copy(deg2b, deg2_h.at[w_id])
    pltpu.sync_copy(s2b, s2_h.at[w_id])

    @pl.when(jnp.logical_and(c == 0, s == 0))
    def _():
        pltpu.async_copy(x_h.at[n0b.at[pl.ds(0, NG)]], rowbuf.at[pl.ds(0, NG)],
                         sem).wait()
        pltpu.sync_copy(rowbuf.at[pl.ds(0, NG)], xroot_h)
        pltpu.async_copy(xs2_h.at[n0b.at[pl.ds(0, NG)]], rowbuf.at[pl.ds(0, NG)],
                         sem).wait()
        pltpu.sync_copy(rowbuf.at[pl.ds(0, NG)], xroot2_h)
        z16f_ = jnp.zeros((L,), _f32)
        for k in range(NG // L):
            idxv = n0b[pl.ds(k * L, L)]
            rootb[pl.ds(k * L, L)] = plsc.load_gather(asrc_t, [idxv])
        for k in range(NG // L, NGP // L):
            rootb[pl.ds(k * L, L)] = z16f_
        pltpu.sync_copy(rootb, asroot_h)
        for k in range(NG // L):
            idxv = n0b[pl.ds(k * L, L)]
            rootb[pl.ds(k * L, L)] = plsc.load_gather(adst_t, [idxv])
        pltpu.sync_copy(rootb, adroot_h)
        for k in range(NG // L):
            idxv = n0b[pl.ds(k * L, L)]
            mv = plsc.load_gather(member_t, [idxv])
            n0b[pl.ds(k * L, L)] = mv
        pltpu.sync_copy(n0b, remap_h)


def _sc_layer2(src, dst, ae2, asrc2, adst2, batch, x, xs2):
    mesh = plsc.VectorSubcoreMesh(core_axis_name="c", subcore_axis_name="s")
    fn = pl.kernel(
        _sc_layer2_body,
        out_type=[
            jax.ShapeDtypeStruct((NW, NG, C), _f32),
            jax.ShapeDtypeStruct((NW, NGP), _f32),
            jax.ShapeDtypeStruct((NW, NGP), _f32),
            jax.ShapeDtypeStruct((NW, NGP), _f32),
            jax.ShapeDtypeStruct((NG, C), _f32),
            jax.ShapeDtypeStruct((NGP,), _f32),
            jax.ShapeDtypeStruct((NGP,), _f32),
            jax.ShapeDtypeStruct((NGP,), _i32),
            jax.ShapeDtypeStruct((NG, C), _f32),
        ],
        mesh=mesh,
        scratch_types=[
            pltpu.VMEM((NP,), _i32),
            pltpu.VMEM((NP,), _f32),
            pltpu.VMEM((NP,), _f32),
            pltpu.VMEM((SEL,), _i32),
            pltpu.VMEM((SEL,), _i32),
            pltpu.VMEM((SEL,), _f32),
            pltpu.VMEM((KS,), _i32),
            pltpu.VMEM((KS,), _i32),
            pltpu.VMEM((KS,), _f32),
            pltpu.VMEM((K, C), _f32),
            pltpu.VMEM((NG, C), _f32),
            pltpu.VMEM((NGP,), _f32),
            pltpu.VMEM((NGP,), _f32),
            pltpu.VMEM((NGP,), _f32),
            pltpu.VMEM((NGP,), _i32),
            pltpu.VMEM((NGP,), _f32),
            pltpu.VMEM_SHARED((NP,), _i32),
            pltpu.SemaphoreType.DMA,
        ],
        compiler_params=pltpu.CompilerParams(needs_layout_passes=False),
    )
    return fn(src, dst, ae2, asrc2, adst2, batch, x, xs2)


def _heads_body(acc2_ref, den2_ref, deg2_ref, s2_ref, asr_ref, adr_ref,
                remap_ref, xr2_ref, xroot_ref, b2_ref, sw_ref, sb_ref,
                w1_ref, b1c_ref, w2_ref, b2c_ref, out_ref):
    num2 = jnp.sum(acc2_ref[...], axis=0)
    den2 = jnp.sum(den2_ref[...], axis=0)[:NG]
    deg2 = jnp.sum(deg2_ref[...], axis=0)[:NG]
    s2 = jnp.sum(s2_ref[...], axis=0)[:NG]
    asr = asr_ref[0, :NG]
    adr = adr_ref[0, :NG]
    al = asr + adr + s2 / jnp.maximum(deg2, 1.0)
    wl = jnp.exp(jnp.where(al >= 0.0, al, 0.2 * al))
    xr2 = xr2_ref[...]
    h2s = (num2 + wl[:, None] * xr2) / (den2 + wl)[:, None] + b2_ref[0]
    remap = remap_ref[:NG]
    onehot = (lax.broadcasted_iota(_i32, (NG, NG), 1) == remap).astype(_f32)
    h2 = jnp.dot(onehot, h2s, preferred_element_type=_f32)
    gin = jnp.concatenate([h2, xroot_ref[:, :ED]], axis=1)
    g = jnp.dot(gin, sw_ref[...], preferred_element_type=_f32) + sb_ref[0]
    g = jnp.maximum(g, 0.0)
    t = jnp.dot(g, w1_ref[...], preferred_element_type=_f32) + b1c_ref[0]
    t = jnp.maximum(t, 0.0)
    out_ref[...] = jnp.dot(t, w2_ref[...], preferred_element_type=_f32) + b2c_ref[0]


def _heads(acc2_p, den2_p, deg2_p, s2_p, asroot, adroot, remap, xroot2, xroot,
           b2, sW, sb, W1c, b1c, W2bd, b2c):
    nh = len(ODIMS)
    return pl.pallas_call(
        _heads_body,
        out_shape=jax.ShapeDtypeStruct((NG, nh * OPAD), _f32),
    )(acc2_p, den2_p, deg2_p, s2_p, asroot, adroot, remap, xroot2, xroot,
      b2, sW, sb, W1c, b1c, W2bd, b2c)


@jax.jit
def kernel(x, edge_index, batch, edge_attr, params):
    src = edge_index[0].astype(_i32)
    dst = edge_index[1].astype(_i32)
    batch_p = jnp.concatenate(
        [batch.astype(_i32), jnp.full((NP - N,), NG - 1, _i32)])
    x_p = jnp.zeros((NP, F), _f32).at[:N].set(x)
    g1, g2 = params['gat']

    ve1 = g1['W_edge'] @ g1['att_edge'][0]
    ve2 = g2['W_edge'] @ g2['att_edge'][0]
    V8 = jnp.zeros((ED, 8), _f32).at[:, 0].set(ve1).at[:, 1].set(ve2)
    A1 = jnp.zeros((C, 8), _f32).at[:, 0].set(g1['att_src'][0]).at[:, 1].set(g1['att_dst'][0])
    A2 = jnp.zeros((C, 8), _f32).at[:, 0].set(g2['att_src'][0]).at[:, 1].set(g2['att_dst'][0])

    ae8 = _eproj(edge_attr, V8, 8000)
    ae1 = ae8[:, 0]
    ae2 = ae8[:, 1]

    xs1, asd1 = _proj(x_p, g1['W'], A1)
    asrc1 = asd1[:, 0]
    adst1 = asd1[:, 1]

    num_p, den_p, deg_p, s1_p = _sc_layer1(src, dst, ae1, asrc1, adst1, xs1)

    b1 = g1['bias'].reshape(1, C)
    xs2, asd2 = _combine(num_p, den_p, deg_p, s1_p, asd1, xs1, g2['W'], A2, b1)
    asrc2 = asd2[:, 0]
    adst2 = asd2[:, 1]

    (acc2_p, den2_p, deg2_p, s2_p, xroot2, asroot, adroot, remap,
     xroot) = _sc_layer2(src, dst, ae2, asrc2, adst2, batch_p, x_p, xs2)

    names = ['ptr_l1', 'ptr_l2', 'ptr_l3', 'leaf_category', 'leaf_signed',
             'leaf_floating', 'leaf_size']
    nh = len(names)
    W1c = jnp.concatenate([params['heads'][n][0][0] for n in names], axis=1)
    b1c = jnp.concatenate([params['heads'][n][0][1] for n in names]).reshape(1, -1)
    W2bd = jnp.zeros((nh * TASK_HIDDEN, nh * OPAD), _f32)
    b2c = jnp.zeros((1, nh * OPAD), _f32)
    for i, n in enumerate(names):
        W2h, b2h = params['heads'][n][1]
        od = ODIMS[i]
        W2bd = W2bd.at[i * TASK_HIDDEN:(i + 1) * TASK_HIDDEN,
                       i * OPAD:i * OPAD + od].set(W2h)
        b2c = b2c.at[0, i * OPAD:i * OPAD + od].set(b2h)
    (sW, sb) = params['shared'][0]

    out = _heads(acc2_p, den2_p, deg2_p, s2_p, asroot.reshape(1, NGP),
                 adroot.reshape(1, NGP), remap.reshape(NGP, 1),
                 xroot2, xroot, g2['bias'].reshape(1, C), sW,
                 sb.reshape(1, -1), W1c, b1c, W2bd, b2c)

    return tuple(out[:, i * OPAD:i * OPAD + od] for i, od in enumerate(ODIMS))
